# baseline (speedup 1.0000x reference)
import functools

import jax
import jax.numpy as jnp
from jax import lax
from jax.experimental import pallas as pl
from jax.experimental.pallas import tpu as pltpu
from jax.experimental.pallas import tpu_sc as plsc

N_NODES = 10000
N_EDGES = 320000
NPAD = 10240
NC = 2
NS = 16
L = 16
NPT = NPAD // NS
CPT = NPAD * 16 // NS
CH = 2000
SUB = 80
NSUB = CH // SUB
DEG_EPT = N_EDGES // NS
DEG_CHUNKS = DEG_EPT // CH
SCAT_EPS = N_EDGES // NC
SCAT_EPT = SCAT_EPS // NS
SCAT_CHUNKS = SCAT_EPT // CH


def _rsqrt16(d):
    bi = plsc.bitcast(d, jnp.int32)
    y = plsc.bitcast(jnp.int32(0x5F3759DF) - lax.shift_right_arithmetic(bi, 1),
                     jnp.float32)
    for _ in range(3):
        y = y * (1.5 - 0.5 * d * y * y)
    return y


def _sc_body(row_h, col_h, a0_h, a1_h, cpart0_h, cpart1_h,
             rbuf, cbuf, a0buf, a1buf, normbuf, flatbuf, ones,
             sqbuf, disbuf, big, sem, semb, deg_sh, degstream_sh, dis_sh,
             c_sh):
    c = lax.axis_index("c")
    s = lax.axis_index("s")

    zero16 = jnp.zeros((L,), jnp.float32)
    iota16 = jnp.arange(L, dtype=jnp.int32)
    zidx16 = jnp.zeros((L,), jnp.int32)

    def fill_zero(i, _):
        big[pl.ds(i * L, L)] = zero16
        return 0

    lax.fori_loop(0, NPAD // L, fill_zero, 0)

    def fill_ones(i, _):
        ones[pl.ds(i * L, L)] = jnp.ones((L,), jnp.float32)
        return 0

    lax.fori_loop(0, SUB // L, fill_ones, 0)

    pltpu.sync_copy(big, c_sh.at[pl.ds(s * CPT, CPT)])
    pltpu.sync_copy(big.at[pl.ds(0, NPT)],
                    degstream_sh.at[pl.ds(s * NPT, NPT)])
    plsc.subcore_barrier()

    base1 = s * DEG_EPT

    def hist_groups(buf):
        for g in range(CH // L):
            rv = buf[pl.ds(g * L, L)]
            cnt, lastm = plsc.scan_count(rv)
            plsc.addupdate_scatter(big, [rv], cnt.astype(jnp.float32),
                                   mask=lastm)

    pltpu.sync_copy(row_h.at[pl.ds(base1, CH)], rbuf)
    pltpu.sync_copy(row_h.at[pl.ds(base1 + CH, CH)], cbuf)

    def deg_pair(i, _):
        i0 = 2 * i
        for j in range(NSUB):
            for g in range(SUB // L):
                flatbuf[j, pl.ds(g * L, L)] = cbuf[pl.ds(j * SUB + g * L, L)]
        descs = []
        for j in range(NSUB):
            descs.append(pltpu.async_copy(
                ones, degstream_sh.at[flatbuf.at[j]], semb, add=True))

        @pl.when(i + 1 < DEG_CHUNKS // 2)
        def _():
            pltpu.async_copy(
                row_h.at[pl.ds(base1 + (i0 + 3) * CH, CH)], cbuf, sem)

        hist_groups(rbuf)

        @pl.when(i + 1 < DEG_CHUNKS // 2)
        def _():
            pltpu.async_copy(
                row_h.at[pl.ds(base1 + (i0 + 2) * CH, CH)], rbuf, sem)

        for d in descs:
            d.wait()

        @pl.when(i + 1 < DEG_CHUNKS // 2)
        def _():
            pltpu.make_async_copy(
                row_h.at[pl.ds(base1 + (i0 + 3) * CH, CH)], cbuf, sem).wait()
            pltpu.make_async_copy(
                row_h.at[pl.ds(base1 + (i0 + 2) * CH, CH)], rbuf, sem).wait()
        return 0

    lax.fori_loop(0, DEG_CHUNKS // 2, deg_pair, 0)

    pltpu.sync_copy(big, deg_sh.at[s])
    plsc.subcore_barrier()

    nbase = s * NPT
    descs = []
    for t in range(NS):
        descs.append(pltpu.async_copy(deg_sh.at[t, pl.ds(nbase, NPT)],
                                      big.at[pl.ds(t * NPT, NPT)], semb))
    descs.append(pltpu.async_copy(degstream_sh.at[pl.ds(nbase, NPT)],
                                  disbuf, semb))
    for d in descs:
        d.wait()

    def combine(g, _):
        acc = big[pl.ds(g * L, L)] + disbuf[pl.ds(g * L, L)]
        for t in range(1, NS):
            acc = acc + big[pl.ds(t * NPT + g * L, L)]
        r = _rsqrt16(acc + 1.0)
        disbuf[pl.ds(g * L, L)] = r
        sqbuf[pl.ds(g * L, L)] = r * r
        return 0

    lax.fori_loop(0, NPT // L, combine, 0)

    pltpu.sync_copy(disbuf, dis_sh.at[pl.ds(nbase, NPT)])
    plsc.subcore_barrier()

    pltpu.sync_copy(dis_sh, big)

    base2 = c * SCAT_EPS + s * SCAT_EPT

    def stage_chunk(i, asynchronous):
        off = base2 + i * CH
        copies = [(row_h.at[pl.ds(off, CH)], rbuf),
                  (col_h.at[pl.ds(off, CH)], cbuf),
                  (a0_h.at[pl.ds(off, CH)], a0buf),
                  (a1_h.at[pl.ds(off, CH)], a1buf)]
        if asynchronous:
            for src, dst in copies:
                pltpu.async_copy(src, dst, sem)
        else:
            for src, dst in copies:
                pltpu.make_async_copy(src, dst, sem).wait()

    stage_chunk(0, True)

    def scat_chunk(i, _):
        stage_chunk(i, False)

        for j in range(NSUB):
            for g in range(SUB // L):
                p = j * SUB + g * L
                rv = rbuf[pl.ds(p, L)]
                cv = cbuf[pl.ds(p, L)]
                av0 = a0buf[pl.ds(p, L)]
                av1 = a1buf[pl.ds(p, L)]
                dr = plsc.load_gather(big, [rv])
                dc = plsc.load_gather(big, [cv])
                normbuf[j, pl.ds(g * L, L)] = dr * dc
                flatbuf[j, pl.ds(g * L, L)] = cv * 16 + av0 * 3 + av1

        descs = []
        for j in range(NSUB):
            descs.append(pltpu.async_copy(
                normbuf.at[j], c_sh.at[flatbuf.at[j]], semb, add=True))

        @pl.when(i + 1 < SCAT_CHUNKS)
        def _():
            stage_chunk(i + 1, True)

        for d in descs:
            d.wait()
        return 0

    lax.fori_loop(0, SCAT_CHUNKS, scat_chunk, 0)
    plsc.subcore_barrier()

    pltpu.sync_copy(c_sh.at[pl.ds(s * CPT, CPT)], big)

    @pl.when(c == 0)
    def _():
        def inject(g, _):
            idx = iota16 * 16 + (256 * g + 9)
            plsc.store_scatter(big, [idx], sqbuf[pl.ds(g * L, L)])
            return 0

        lax.fori_loop(0, NPT // L, inject, 0)
        pltpu.sync_copy(big, cpart0_h.at[pl.ds(s * CPT, CPT)])

    @pl.when(c == 1)
    def _():
        pltpu.sync_copy(big, cpart1_h.at[pl.ds(s * CPT, CPT)])


_sc_call = pl.kernel(
    _sc_body,
    out_type=(
        jax.ShapeDtypeStruct((NPAD * 16,), jnp.float32),
        jax.ShapeDtypeStruct((NPAD * 16,), jnp.float32),
    ),
    mesh=plsc.VectorSubcoreMesh(core_axis_name="c", subcore_axis_name="s"),
    compiler_params=pltpu.CompilerParams(needs_layout_passes=False),
    scratch_types=(
        pltpu.VMEM((CH,), jnp.int32),
        pltpu.VMEM((CH,), jnp.int32),
        pltpu.VMEM((CH,), jnp.int32),
        pltpu.VMEM((CH,), jnp.int32),
        pltpu.VMEM((NSUB, SUB), jnp.float32),
        pltpu.VMEM((NSUB, SUB), jnp.int32),
        pltpu.VMEM((SUB,), jnp.float32),
        pltpu.VMEM((NPT,), jnp.float32),
        pltpu.VMEM((NPT,), jnp.float32),
        pltpu.VMEM((NPAD,), jnp.float32),
        pltpu.SemaphoreType.DMA,
        pltpu.SemaphoreType.DMA,
        pltpu.VMEM_SHARED((NS, NPAD), jnp.float32),
        pltpu.VMEM_SHARED((NPAD,), jnp.float32),
        pltpu.VMEM_SHARED((NPAD,), jnp.float32),
        pltpu.VMEM_SHARED((NPAD * 16,), jnp.float32),
    ),
    name="ligand_edge_tables_sc",
)

BR = 5000
G = N_NODES // BR


def _tc_a_body(enc_ref, wv_ref, bv_ref, wb_ref, bb_ref, node_ref, bpad_ref):
    enc = enc_ref[...]
    h = jnp.dot(enc, wv_ref[...],
                preferred_element_type=jnp.float32) + bv_ref[...]
    node_ref[...] = jnp.broadcast_to(h, node_ref.shape)

    bondrow = jnp.dot(2.0 * h, wb_ref[...],
                      preferred_element_type=jnp.float32) + bb_ref[...]
    l_iota = lax.broadcasted_iota(jnp.int32, (8, 128), 1)
    acc = jnp.zeros((8, 128), jnp.float32)
    for j in range(5):
        acc = jnp.where(l_iota == j, bondrow[0, j], acc)
    bpad_ref[...] = acc


def _tc_b_body(enc_ref, aprelu_ref, cprelu_ref, wv_ref, bv_ref,
               awe_ref, awc_ref, abc_ref, aee1_ref, aee2_ref,
               cwe_ref, cwc_ref, cbc_ref, cee1_ref, cee2_ref,
               c0_ref, c1_ref, atom_ref, chi_ref):
    enc = enc_ref[...]
    h = jnp.dot(enc, wv_ref[...],
                preferred_element_type=jnp.float32) + bv_ref[...]
    cfull = c0_ref[...] + c1_ref[...]

    def decoder(a, we_ref, wc_ref, bc_ref, ee1_ref, ee2_ref, out_ref):
        p = jnp.where(h >= 0, h, a * h)
        d = jnp.dot(p, we_ref[...], preferred_element_type=jnp.float32)
        y = jnp.dot(d, wc_ref[...],
                    preferred_element_type=jnp.float32) + bc_ref[...]
        ee1 = ee1_ref[...]
        ee2 = ee2_ref[...]
        F = y.shape[1]
        e1rep = jnp.reshape(jnp.broadcast_to(ee1[0:3][:, None, :], (3, 3, F)),
                            (9, F))
        e2til = jnp.reshape(jnp.broadcast_to(ee2[None, 0:3, :], (3, 3, F)),
                            (9, F))
        m = jnp.concatenate(
            [e1rep + e2til, ee1[4:5] + ee2[0:1], jnp.zeros((6, F), jnp.float32)],
            axis=0) + y
        out_ref[...] = jnp.dot(cfull, m, preferred_element_type=jnp.float32,
                               precision=lax.Precision.HIGHEST)

    decoder(aprelu_ref[0, 0], awe_ref, awc_ref, abc_ref, aee1_ref, aee2_ref,
            atom_ref)
    decoder(cprelu_ref[0, 0], cwe_ref, cwc_ref, cbc_ref, cee1_ref, cee2_ref,
            chi_ref)


def _full(shape):
    return pl.BlockSpec(shape, lambda i: (0,) * len(shape))


_tc_a_call = pl.pallas_call(
    _tc_a_body,
    grid=(G,),
    in_specs=[
        _full((1, 128)),
        _full((128, 128)),
        _full((1, 128)),
        _full((128, 5)),
        _full((1, 5)),
    ],
    out_specs=[
        pl.BlockSpec((BR, 128), lambda i: (i, 0)),
        _full((8, 128)),
    ],
    out_shape=[
        jax.ShapeDtypeStruct((N_NODES, 128), jnp.float32),
        jax.ShapeDtypeStruct((8, 128), jnp.float32),
    ],
    name="ligand_node_bond_tc",
)

_tc_b_call = pl.pallas_call(
    _tc_b_body,
    grid=(G,),
    in_specs=[
        _full((1, 128)),
        _full((1, 1)),
        _full((1, 1)),
        _full((128, 128)),
        _full((1, 128)),
        _full((128, 128)),
        _full((128, 119)),
        _full((1, 119)),
        _full((6, 119)),
        _full((3, 119)),
        _full((128, 128)),
        _full((128, 5)),
        _full((1, 5)),
        _full((6, 5)),
        _full((3, 5)),
        pl.BlockSpec((BR, 16), lambda i: (i, 0)),
        pl.BlockSpec((BR, 16), lambda i: (i, 0)),
    ],
    out_specs=[
        pl.BlockSpec((BR, 119), lambda i: (i, 0)),
        pl.BlockSpec((BR, 5), lambda i: (i, 0)),
    ],
    out_shape=[
        jax.ShapeDtypeStruct((N_NODES, 119), jnp.float32),
        jax.ShapeDtypeStruct((N_NODES, 5), jnp.float32),
    ],
    name="ligand_expand_tc",
)


def kernel(encoded_vectors, edge_index, edge_attr, num_nodes, W_v2n, b_v2n,
           atom_prelu, atom_We2d, atom_Wc, atom_bc, atom_ee1, atom_ee2,
           chi_prelu, chi_We2d, chi_Wc, chi_bc, chi_ee1, chi_ee2,
           W_bond, b_bond):
    ea = edge_attr.T
    cpart0, cpart1 = _sc_call(edge_index[0], edge_index[1], ea[0], ea[1])

    node, bpad = _tc_a_call(
        encoded_vectors,
        W_v2n,
        jnp.reshape(b_v2n, (1, 128)),
        W_bond,
        jnp.reshape(b_bond, (1, 5)),
    )
    bond = jnp.broadcast_to(bpad[0:1, 0:5], (N_EDGES, 5))

    atom, chi = _tc_b_call(
        encoded_vectors,
        jnp.reshape(atom_prelu.astype(jnp.float32), (1, 1)),
        jnp.reshape(chi_prelu.astype(jnp.float32), (1, 1)),
        W_v2n,
        jnp.reshape(b_v2n, (1, 128)),
        atom_We2d, atom_Wc,
        jnp.reshape(atom_bc, (1, 119)),
        atom_ee1, atom_ee2,
        chi_We2d, chi_Wc,
        jnp.reshape(chi_bc, (1, 5)),
        chi_ee1, chi_ee2,
        cpart0.reshape(NPAD, 16),
        cpart1.reshape(NPAD, 16),
    )
    return (atom, chi, bond, node)


# speedup vs baseline: 1.0729x; 1.0729x over previous
import functools

import jax
import jax.numpy as jnp
from jax import lax
from jax.experimental import pallas as pl
from jax.experimental.pallas import tpu as pltpu
from jax.experimental.pallas import tpu_sc as plsc

N_NODES = 10000
N_EDGES = 320000
NPAD = 10240
NC = 2
NS = 16
L = 16
NPT = NPAD // NS
CPT = NPAD * 16 // NS
CH = 2000
SUB = 80
NSUB = CH // SUB
DEG_EPT = N_EDGES // NS
DEG_CHUNKS = DEG_EPT // CH
SCAT_EPS = N_EDGES // NC
SCAT_EPT = SCAT_EPS // NS
SCAT_CHUNKS = SCAT_EPT // CH


def _rsqrt16(d):
    bi = plsc.bitcast(d, jnp.int32)
    y = plsc.bitcast(jnp.int32(0x5F3759DF) - lax.shift_right_arithmetic(bi, 1),
                     jnp.float32)
    for _ in range(3):
        y = y * (1.5 - 0.5 * d * y * y)
    return y


def _sc_body(row_h, col_h, a0_h, a1_h, cpart0_h, cpart1_h,
             rbuf, cbuf, a0buf, a1buf, normbuf, flatbuf, ones,
             sqbuf, disbuf, big, sem, semb, deg_sh, degstream_sh, dis_sh,
             c_sh):
    c = lax.axis_index("c")
    s = lax.axis_index("s")

    zero16 = jnp.zeros((L,), jnp.float32)
    iota16 = jnp.arange(L, dtype=jnp.int32)
    zidx16 = jnp.zeros((L,), jnp.int32)

    def fill_zero(i, _):
        big[pl.ds(i * L, L)] = zero16
        return 0

    lax.fori_loop(0, NPAD // L, fill_zero, 0)

    def fill_ones(i, _):
        ones[pl.ds(i * L, L)] = jnp.ones((L,), jnp.float32)
        return 0

    lax.fori_loop(0, SUB // L, fill_ones, 0)

    pltpu.sync_copy(big, c_sh.at[pl.ds(s * CPT, CPT)])
    pltpu.sync_copy(big.at[pl.ds(0, NPT)],
                    degstream_sh.at[pl.ds(s * NPT, NPT)])
    plsc.subcore_barrier()

    base1 = s * DEG_EPT

    def hist_groups(buf):
        for g in range(CH // L):
            rv = buf[pl.ds(g * L, L)]
            cnt, lastm = plsc.scan_count(rv)
            plsc.addupdate_scatter(big, [rv], cnt.astype(jnp.float32),
                                   mask=lastm)

    pltpu.sync_copy(row_h.at[pl.ds(base1, CH)], rbuf)
    pltpu.sync_copy(row_h.at[pl.ds(base1 + CH, CH)], cbuf)

    def deg_pair(i, _):
        i0 = 2 * i
        for j in range(NSUB):
            for g in range(SUB // L):
                flatbuf[j, pl.ds(g * L, L)] = cbuf[pl.ds(j * SUB + g * L, L)]
        descs = []
        for j in range(NSUB):
            descs.append(pltpu.async_copy(
                ones, degstream_sh.at[flatbuf.at[j]], semb, add=True))

        @pl.when(i + 1 < DEG_CHUNKS // 2)
        def _():
            pltpu.async_copy(
                row_h.at[pl.ds(base1 + (i0 + 3) * CH, CH)], cbuf, sem)

        hist_groups(rbuf)

        @pl.when(i + 1 < DEG_CHUNKS // 2)
        def _():
            pltpu.async_copy(
                row_h.at[pl.ds(base1 + (i0 + 2) * CH, CH)], rbuf, sem)

        for d in descs:
            d.wait()

        @pl.when(i + 1 < DEG_CHUNKS // 2)
        def _():
            pltpu.make_async_copy(
                row_h.at[pl.ds(base1 + (i0 + 3) * CH, CH)], cbuf, sem).wait()
            pltpu.make_async_copy(
                row_h.at[pl.ds(base1 + (i0 + 2) * CH, CH)], rbuf, sem).wait()
        return 0

    lax.fori_loop(0, DEG_CHUNKS // 2, deg_pair, 0)

    pltpu.sync_copy(big, deg_sh.at[s])
    plsc.subcore_barrier()

    nbase = s * NPT
    descs = []
    for t in range(NS):
        descs.append(pltpu.async_copy(deg_sh.at[t, pl.ds(nbase, NPT)],
                                      big.at[pl.ds(t * NPT, NPT)], semb))
    descs.append(pltpu.async_copy(degstream_sh.at[pl.ds(nbase, NPT)],
                                  disbuf, semb))
    for d in descs:
        d.wait()

    def combine(g, _):
        acc = big[pl.ds(g * L, L)] + disbuf[pl.ds(g * L, L)]
        for t in range(1, NS):
            acc = acc + big[pl.ds(t * NPT + g * L, L)]
        r = _rsqrt16(acc + 1.0)
        disbuf[pl.ds(g * L, L)] = r
        sqbuf[pl.ds(g * L, L)] = r * r
        return 0

    lax.fori_loop(0, NPT // L, combine, 0)

    pltpu.sync_copy(disbuf, dis_sh.at[pl.ds(nbase, NPT)])
    plsc.subcore_barrier()

    pltpu.sync_copy(dis_sh, big)

    base2 = c * SCAT_EPS + s * SCAT_EPT

    def stage_chunk(i, asynchronous):
        off = base2 + i * CH
        copies = [(row_h.at[pl.ds(off, CH)], rbuf),
                  (col_h.at[pl.ds(off, CH)], cbuf),
                  (a0_h.at[pl.ds(off, CH)], a0buf),
                  (a1_h.at[pl.ds(off, CH)], a1buf)]
        if asynchronous:
            for src, dst in copies:
                pltpu.async_copy(src, dst, sem)
        else:
            for src, dst in copies:
                pltpu.make_async_copy(src, dst, sem).wait()

    stage_chunk(0, True)

    def scat_chunk(i, _):
        stage_chunk(i, False)

        for j in range(NSUB):
            for g in range(SUB // L):
                p = j * SUB + g * L
                rv = rbuf[pl.ds(p, L)]
                cv = cbuf[pl.ds(p, L)]
                av0 = a0buf[pl.ds(p, L)]
                av1 = a1buf[pl.ds(p, L)]
                dr = plsc.load_gather(big, [rv])
                dc = plsc.load_gather(big, [cv])
                normbuf[j, pl.ds(g * L, L)] = dr * dc
                flatbuf[j, pl.ds(g * L, L)] = cv * 16 + av0 * 3 + av1

        descs = []
        for j in range(NSUB):
            descs.append(pltpu.async_copy(
                normbuf.at[j], c_sh.at[flatbuf.at[j]], semb, add=True))

        @pl.when(i + 1 < SCAT_CHUNKS)
        def _():
            stage_chunk(i + 1, True)

        for d in descs:
            d.wait()
        return 0

    lax.fori_loop(0, SCAT_CHUNKS, scat_chunk, 0)
    plsc.subcore_barrier()

    pltpu.sync_copy(c_sh.at[pl.ds(s * CPT, CPT)], big)

    @pl.when(c == 0)
    def _():
        def inject(g, _):
            idx = iota16 * 16 + (256 * g + 9)
            plsc.store_scatter(big, [idx], sqbuf[pl.ds(g * L, L)])
            return 0

        lax.fori_loop(0, NPT // L, inject, 0)
        pltpu.sync_copy(big, cpart0_h.at[pl.ds(s * CPT, CPT)])

    @pl.when(c == 1)
    def _():
        pltpu.sync_copy(big, cpart1_h.at[pl.ds(s * CPT, CPT)])


_sc_call = pl.kernel(
    _sc_body,
    out_type=(
        jax.ShapeDtypeStruct((NPAD * 16,), jnp.float32),
        jax.ShapeDtypeStruct((NPAD * 16,), jnp.float32),
    ),
    mesh=plsc.VectorSubcoreMesh(core_axis_name="c", subcore_axis_name="s"),
    compiler_params=pltpu.CompilerParams(needs_layout_passes=False),
    scratch_types=(
        pltpu.VMEM((CH,), jnp.int32),
        pltpu.VMEM((CH,), jnp.int32),
        pltpu.VMEM((CH,), jnp.int32),
        pltpu.VMEM((CH,), jnp.int32),
        pltpu.VMEM((NSUB, SUB), jnp.float32),
        pltpu.VMEM((NSUB, SUB), jnp.int32),
        pltpu.VMEM((SUB,), jnp.float32),
        pltpu.VMEM((NPT,), jnp.float32),
        pltpu.VMEM((NPT,), jnp.float32),
        pltpu.VMEM((NPAD,), jnp.float32),
        pltpu.SemaphoreType.DMA,
        pltpu.SemaphoreType.DMA,
        pltpu.VMEM_SHARED((NS, NPAD), jnp.float32),
        pltpu.VMEM_SHARED((NPAD,), jnp.float32),
        pltpu.VMEM_SHARED((NPAD,), jnp.float32),
        pltpu.VMEM_SHARED((NPAD * 16,), jnp.float32),
    ),
    name="ligand_edge_tables_sc",
)

BR = 2000
G = N_NODES // BR


def _tc_a_body(enc_ref, wv_ref, bv_ref, wb_ref, bb_ref, node_ref, bpad_ref):
    enc = enc_ref[...]
    h = jnp.dot(enc, wv_ref[...],
                preferred_element_type=jnp.float32) + bv_ref[...]
    node_ref[...] = jnp.broadcast_to(h, node_ref.shape)

    bondrow = jnp.dot(2.0 * h, wb_ref[...],
                      preferred_element_type=jnp.float32) + bb_ref[...]
    l_iota = lax.broadcasted_iota(jnp.int32, (8, 128), 1)
    acc = jnp.zeros((8, 128), jnp.float32)
    for j in range(5):
        acc = jnp.where(l_iota == j, bondrow[0, j], acc)
    bpad_ref[...] = acc


def _tc_b_body(enc_ref, aprelu_ref, cprelu_ref, wv_ref, bv_ref,
               awe_ref, awc_ref, abc_ref, aee1_ref, aee2_ref,
               cwe_ref, cwc_ref, cbc_ref, cee1_ref, cee2_ref,
               c0_ref, c1_ref, atom_ref, chi_ref):
    enc = enc_ref[...]
    h = jnp.dot(enc, wv_ref[...],
                preferred_element_type=jnp.float32) + bv_ref[...]
    cfull = c0_ref[...] + c1_ref[...]

    def decoder(a, we_ref, wc_ref, bc_ref, ee1_ref, ee2_ref, out_ref):
        p = jnp.where(h >= 0, h, a * h)
        d = jnp.dot(p, we_ref[...], preferred_element_type=jnp.float32)
        y = jnp.dot(d, wc_ref[...],
                    preferred_element_type=jnp.float32) + bc_ref[...]
        ee1 = ee1_ref[...]
        ee2 = ee2_ref[...]
        F = y.shape[1]
        e1rep = jnp.reshape(jnp.broadcast_to(ee1[0:3][:, None, :], (3, 3, F)),
                            (9, F))
        e2til = jnp.reshape(jnp.broadcast_to(ee2[None, 0:3, :], (3, 3, F)),
                            (9, F))
        m = jnp.concatenate(
            [e1rep + e2til, ee1[4:5] + ee2[0:1], jnp.zeros((6, F), jnp.float32)],
            axis=0) + y
        out_ref[...] = jnp.dot(cfull, m, preferred_element_type=jnp.float32,
                               precision=lax.Precision.HIGHEST)

    decoder(aprelu_ref[0, 0], awe_ref, awc_ref, abc_ref, aee1_ref, aee2_ref,
            atom_ref)
    decoder(cprelu_ref[0, 0], cwe_ref, cwc_ref, cbc_ref, cee1_ref, cee2_ref,
            chi_ref)


def _full(shape):
    return pl.BlockSpec(shape, lambda i: (0,) * len(shape))


_tc_a_call = pl.pallas_call(
    _tc_a_body,
    grid=(G,),
    in_specs=[
        _full((1, 128)),
        _full((128, 128)),
        _full((1, 128)),
        _full((128, 5)),
        _full((1, 5)),
    ],
    out_specs=[
        pl.BlockSpec((BR, 128), lambda i: (i, 0)),
        _full((8, 128)),
    ],
    out_shape=[
        jax.ShapeDtypeStruct((N_NODES, 128), jnp.float32),
        jax.ShapeDtypeStruct((8, 128), jnp.float32),
    ],
    name="ligand_node_bond_tc",
)

_tc_b_call = pl.pallas_call(
    _tc_b_body,
    grid=(G,),
    in_specs=[
        _full((1, 128)),
        _full((1, 1)),
        _full((1, 1)),
        _full((128, 128)),
        _full((1, 128)),
        _full((128, 128)),
        _full((128, 119)),
        _full((1, 119)),
        _full((6, 119)),
        _full((3, 119)),
        _full((128, 128)),
        _full((128, 5)),
        _full((1, 5)),
        _full((6, 5)),
        _full((3, 5)),
        pl.BlockSpec((BR, 16), lambda i: (i, 0)),
        pl.BlockSpec((BR, 16), lambda i: (i, 0)),
    ],
    out_specs=[
        pl.BlockSpec((BR, 119), lambda i: (i, 0)),
        pl.BlockSpec((BR, 5), lambda i: (i, 0)),
    ],
    out_shape=[
        jax.ShapeDtypeStruct((N_NODES, 119), jnp.float32),
        jax.ShapeDtypeStruct((N_NODES, 5), jnp.float32),
    ],
    name="ligand_expand_tc",
)


def kernel(encoded_vectors, edge_index, edge_attr, num_nodes, W_v2n, b_v2n,
           atom_prelu, atom_We2d, atom_Wc, atom_bc, atom_ee1, atom_ee2,
           chi_prelu, chi_We2d, chi_Wc, chi_bc, chi_ee1, chi_ee2,
           W_bond, b_bond):
    ea = edge_attr.T
    cpart0, cpart1 = _sc_call(edge_index[0], edge_index[1], ea[0], ea[1])

    node, bpad = _tc_a_call(
        encoded_vectors,
        W_v2n,
        jnp.reshape(b_v2n, (1, 128)),
        W_bond,
        jnp.reshape(b_bond, (1, 5)),
    )
    bond = jnp.broadcast_to(bpad[0:1, 0:5], (N_EDGES, 5))

    atom, chi = _tc_b_call(
        encoded_vectors,
        jnp.reshape(atom_prelu.astype(jnp.float32), (1, 1)),
        jnp.reshape(chi_prelu.astype(jnp.float32), (1, 1)),
        W_v2n,
        jnp.reshape(b_v2n, (1, 128)),
        atom_We2d, atom_Wc,
        jnp.reshape(atom_bc, (1, 119)),
        atom_ee1, atom_ee2,
        chi_We2d, chi_Wc,
        jnp.reshape(chi_bc, (1, 5)),
        chi_ee1, chi_ee2,
        cpart0.reshape(NPAD, 16),
        cpart1.reshape(NPAD, 16),
    )
    return (atom, chi, bond, node)
